# baseline (speedup 1.0000x reference)
import functools

import jax
import jax.numpy as jnp
from jax import lax
from jax.experimental import pallas as pl
from jax.experimental.pallas import tpu as pltpu
from jax.experimental.pallas import tpu_sc as plsc

N = 10000
E = 320000
D = 128
NBLOCKS = 3
NCONVS = 2 + 2 * NBLOCKS

NC = 2
NS = 16
NW = NC * NS
WIN = 40
NB = 8
NWIN = 256
NCHUNK = NWIN // NB
EPW = NWIN * WIN
E_PAD = NW * EPW
N_PAD = 10240
RPW = N_PAD // NS
ZROWS = WIN


def _sc_segment_sum_body(support, src3, dst3, out, acc, srcbuf,
                         dstbuf, *bufs):
    rows = bufs[0:NB]
    gsem = bufs[NB:2 * NB]
    ssem = bufs[2 * NB:3 * NB]
    isem0, dsem, wsem = bufs[3 * NB:3 * NB + 3]

    c = lax.axis_index("c")
    s = lax.axis_index("s")
    wid = c * NS + s

    pltpu.async_copy(src3.at[wid, 0], srcbuf.at[0], isem0)
    pltpu.async_copy(dst3.at[wid, 0], dstbuf.at[0], dsem)

    zbuf = rows[0]

    def _zloop(i, carry):
        for j in range(D // 16):
            zbuf[i, pl.ds(j * 16, 16)] = jnp.zeros((16,), jnp.float32)
        return carry

    lax.fori_loop(0, ZROWS, _zloop, 0, unroll=False)
    zdescs = [pltpu.async_copy(
        zbuf, acc.at[pl.ds(s * RPW + r * ZROWS, ZROWS)], wsem)
        for r in range(RPW // ZROWS)]
    for d_ in zdescs:
        d_.wait()
    pltpu.make_async_copy(src3.at[wid, 0], srcbuf.at[0], isem0).wait()
    for b in range(NB):
        pltpu.async_copy(support.at[srcbuf.at[0, b]], rows[b], gsem[b])
    plsc.subcore_barrier()

    def _chunk(g, carry):
        w0 = g * NB
        p = lax.rem(g, 2)
        pn = lax.rem(g + 1, 2)
        pp = lax.rem(g + 1, 2)
        @pl.when(g > 0)
        def _wait_src_idx():
            pltpu.make_async_copy(
                src3.at[wid, g], srcbuf.at[p], isem0).wait()

        pltpu.make_async_copy(
            dst3.at[wid, g], dstbuf.at[p], dsem).wait()

        for b in range(NB):
            w = w0 + b

            @pl.when(g > 0)
            def _drain_and_gather():
                pltpu.make_async_copy(
                    rows[b], acc.at[dstbuf.at[pp, b]], ssem[b]).wait()
                pltpu.async_copy(
                    support.at[srcbuf.at[p, b]], rows[b], gsem[b])

        @pl.when(g + 1 < NCHUNK)
        def _prefetch():
            pltpu.async_copy(
                src3.at[wid, g + 1], srcbuf.at[pn], isem0)
            pltpu.async_copy(
                dst3.at[wid, g + 1], dstbuf.at[pn], dsem)

        for b in range(NB):
            w = w0 + b
            pltpu.make_async_copy(
                support.at[srcbuf.at[p, b]], rows[b], gsem[b]).wait()
            pltpu.async_copy(rows[b], acc.at[dstbuf.at[p, b]], ssem[b],
                             add=True)
        return carry

    lax.fori_loop(0, NCHUNK, _chunk, 0, unroll=False)
    pl_last = (NCHUNK - 1) % 2
    for b in range(NB):
        pltpu.make_async_copy(
            rows[b], acc.at[dstbuf.at[pl_last, b]], ssem[b]).wait()
    plsc.subcore_barrier()

    wdescs = []
    for r in range(RPW // ZROWS):
        row0 = s * RPW + r * ZROWS
        wdescs.append(pltpu.async_copy(
            acc.at[pl.ds(row0, ZROWS)], out.at[c, pl.ds(row0, ZROWS)], wsem))
    for d_ in wdescs:
        d_.wait()


@functools.cache
def _sc_segment_sum_kernel():
    return pl.kernel(
        _sc_segment_sum_body,
        out_type=jax.ShapeDtypeStruct((NC, N_PAD, D), jnp.float32),
        mesh=plsc.VectorSubcoreMesh(core_axis_name="c", subcore_axis_name="s",
                                    num_cores=NC, num_subcores=NS),
        scratch_types=(
            [pltpu.VMEM_SHARED((N_PAD, D), jnp.float32)]
            + [pltpu.VMEM((2, NB, WIN), jnp.int32)] * 2
            + [pltpu.VMEM((WIN, D), jnp.float32)] * NB
            + [pltpu.SemaphoreType.DMA] * (2 * NB + 3)
        ),
    )


def _sc_segment_sum(s, src, dst):
    npad = E_PAD - E
    pad_src = jnp.arange(npad, dtype=jnp.int32) % N
    pad_dst = jnp.arange(npad, dtype=jnp.int32) % (N_PAD - N - 8) + N
    src3 = jnp.concatenate([src, pad_src]).reshape(NW, NCHUNK, NB, WIN)
    dst3 = jnp.concatenate([dst, pad_dst]).reshape(NW, NCHUNK, NB, WIN)
    return _sc_segment_sum_kernel()(s, src3, dst3)


ROWB = 1000


def _tc_mm_body(x_ref, w_ref, o_ref):
    o_ref[...] = jnp.dot(x_ref[...], w_ref[...],
                         preferred_element_type=jnp.float32)


def _tc_mm(x, w):
    return pl.pallas_call(
        _tc_mm_body,
        grid=(N // ROWB,),
        in_specs=[
            pl.BlockSpec((ROWB, D), lambda i: (i, 0)),
            pl.BlockSpec((D, D), lambda i: (0, 0)),
        ],
        out_specs=pl.BlockSpec((ROWB, D), lambda i: (i, 0)),
        out_shape=jax.ShapeDtypeStruct((N, D), jnp.float32),
    )(x, w)


def _tc_combine_body(relu, p_ref, x_ref, wl_ref, b_ref, w_ref, hres_ref,
                     h_ref, s_ref):
    t = (p_ref[0] + p_ref[1]
         + jnp.dot(x_ref[...], wl_ref[...], preferred_element_type=jnp.float32)
         + b_ref[0])
    if relu:
        t = jnp.maximum(t, 0.0)
    if hres_ref is not None:
        t = (hres_ref[...] + t) * 0.5
    h_ref[...] = t
    s_ref[...] = jnp.dot(t, w_ref[...], preferred_element_type=jnp.float32)


def _tc_combine(p, x, wl, b, w_next, h_res):
    has_res = h_res is not None
    body = functools.partial(_tc_combine_body, True)
    if not has_res:
        body = lambda p_, x_, wl_, b_, w_, h_, s_: _tc_combine_body(
            True, p_, x_, wl_, b_, w_, None, h_, s_)
    in_specs = [
        pl.BlockSpec((NC, ROWB, D), lambda i: (0, i, 0)),
        pl.BlockSpec((ROWB, D), lambda i: (i, 0)),
        pl.BlockSpec((D, D), lambda i: (0, 0)),
        pl.BlockSpec((1, D), lambda i: (0, 0)),
        pl.BlockSpec((D, D), lambda i: (0, 0)),
    ]
    args = [p, x, wl, b.reshape(1, D), w_next]
    if has_res:
        in_specs.append(pl.BlockSpec((ROWB, D), lambda i: (i, 0)))
        args.append(h_res)
    return pl.pallas_call(
        body,
        grid=(N // ROWB,),
        in_specs=in_specs,
        out_specs=(pl.BlockSpec((ROWB, D), lambda i: (i, 0)),
                   pl.BlockSpec((ROWB, D), lambda i: (i, 0))),
        out_shape=(jax.ShapeDtypeStruct((N, D), jnp.float32),
                   jax.ShapeDtypeStruct((N, D), jnp.float32)),
    )(*args)


def _tc_final_body(p_ref, x_ref, wl_ref, b_ref, o_ref):
    o_ref[...] = (p_ref[0] + p_ref[1]
                  + jnp.dot(x_ref[...], wl_ref[...],
                            preferred_element_type=jnp.float32)
                  + b_ref[0])


def _tc_final(p, x, wl, b):
    return pl.pallas_call(
        _tc_final_body,
        grid=(N // ROWB,),
        in_specs=[
            pl.BlockSpec((NC, ROWB, D), lambda i: (0, i, 0)),
            pl.BlockSpec((ROWB, D), lambda i: (i, 0)),
            pl.BlockSpec((D, D), lambda i: (0, 0)),
            pl.BlockSpec((1, D), lambda i: (0, 0)),
        ],
        out_specs=pl.BlockSpec((ROWB, D), lambda i: (i, 0)),
        out_shape=jax.ShapeDtypeStruct((N, D), jnp.float32),
    )(p, x, wl, b.reshape(1, D))


def kernel(inputs, edge_index, W, Wl, b):
    src = edge_index[0]
    dst = edge_index[1]

    s = _tc_mm(inputs, W[0])
    p = _sc_segment_sum(s, src, dst)
    h, s = _tc_combine(p, inputs, Wl[0], b[0], W[1], None)

    for i in range(NBLOCKS):
        j = 1 + 2 * i
        blk_in = h
        p = _sc_segment_sum(s, src, dst)
        t, s = _tc_combine(p, h, Wl[j], b[j], W[j + 1], None)
        p = _sc_segment_sum(s, src, dst)
        h, s = _tc_combine(p, t, Wl[j + 1], b[j + 1], W[j + 2], blk_in)

    p = _sc_segment_sum(s, src, dst)
    x_out = _tc_final(p, h, Wl[NCONVS - 1], b[NCONVS - 1])
    return (x_out, h)


# speedup vs baseline: 1.0375x; 1.0375x over previous
import functools

import jax
import jax.numpy as jnp
from jax import lax
from jax.experimental import pallas as pl
from jax.experimental.pallas import tpu as pltpu
from jax.experimental.pallas import tpu_sc as plsc

N = 10000
E = 320000
D = 128
NBLOCKS = 3
NCONVS = 2 + 2 * NBLOCKS

NC = 2
NS = 16
NW = NC * NS
WIN = 80
NB = 4
NWIN = 128
NCHUNK = NWIN // NB
EPW = NWIN * WIN
E_PAD = NW * EPW
N_PAD = 10240
RPW = N_PAD // NS
ZROWS = WIN


def _sc_segment_sum_body(support, src3, dst3, out, acc, srcbuf,
                         dstbuf, *bufs):
    rows = bufs[0:NB]
    gsem = bufs[NB:2 * NB]
    ssem = bufs[2 * NB:3 * NB]
    isem0, dsem, wsem = bufs[3 * NB:3 * NB + 3]

    c = lax.axis_index("c")
    s = lax.axis_index("s")
    wid = c * NS + s

    pltpu.async_copy(src3.at[wid, 0], srcbuf.at[0], isem0)
    pltpu.async_copy(dst3.at[wid, 0], dstbuf.at[0], dsem)

    zbuf = rows[0]

    def _zloop(i, carry):
        for j in range(D // 16):
            zbuf[i, pl.ds(j * 16, 16)] = jnp.zeros((16,), jnp.float32)
        return carry

    lax.fori_loop(0, ZROWS, _zloop, 0, unroll=False)
    zdescs = [pltpu.async_copy(
        zbuf, acc.at[pl.ds(s * RPW + r * ZROWS, ZROWS)], wsem)
        for r in range(RPW // ZROWS)]
    for d_ in zdescs:
        d_.wait()
    pltpu.make_async_copy(src3.at[wid, 0], srcbuf.at[0], isem0).wait()
    for b in range(NB):
        pltpu.async_copy(support.at[srcbuf.at[0, b]], rows[b], gsem[b])
    plsc.subcore_barrier()

    def _chunk(g, carry):
        w0 = g * NB
        p = lax.rem(g, 2)
        pn = lax.rem(g + 1, 2)
        pp = lax.rem(g + 1, 2)
        @pl.when(g > 0)
        def _wait_src_idx():
            pltpu.make_async_copy(
                src3.at[wid, g], srcbuf.at[p], isem0).wait()

        pltpu.make_async_copy(
            dst3.at[wid, g], dstbuf.at[p], dsem).wait()

        for b in range(NB):
            w = w0 + b

            @pl.when(g > 0)
            def _drain_and_gather():
                pltpu.make_async_copy(
                    rows[b], acc.at[dstbuf.at[pp, b]], ssem[b]).wait()
                pltpu.async_copy(
                    support.at[srcbuf.at[p, b]], rows[b], gsem[b])

        @pl.when(g + 1 < NCHUNK)
        def _prefetch():
            pltpu.async_copy(
                src3.at[wid, g + 1], srcbuf.at[pn], isem0)
            pltpu.async_copy(
                dst3.at[wid, g + 1], dstbuf.at[pn], dsem)

        for b in range(NB):
            w = w0 + b
            pltpu.make_async_copy(
                support.at[srcbuf.at[p, b]], rows[b], gsem[b]).wait()
            pltpu.async_copy(rows[b], acc.at[dstbuf.at[p, b]], ssem[b],
                             add=True)
        return carry

    lax.fori_loop(0, NCHUNK, _chunk, 0, unroll=False)
    pl_last = (NCHUNK - 1) % 2
    for b in range(NB):
        pltpu.make_async_copy(
            rows[b], acc.at[dstbuf.at[pl_last, b]], ssem[b]).wait()
    plsc.subcore_barrier()

    wdescs = []
    for r in range(RPW // ZROWS):
        row0 = s * RPW + r * ZROWS
        wdescs.append(pltpu.async_copy(
            acc.at[pl.ds(row0, ZROWS)], out.at[c, pl.ds(row0, ZROWS)], wsem))
    for d_ in wdescs:
        d_.wait()


@functools.cache
def _sc_segment_sum_kernel():
    return pl.kernel(
        _sc_segment_sum_body,
        out_type=jax.ShapeDtypeStruct((NC, N_PAD, D), jnp.float32),
        mesh=plsc.VectorSubcoreMesh(core_axis_name="c", subcore_axis_name="s",
                                    num_cores=NC, num_subcores=NS),
        scratch_types=(
            [pltpu.VMEM_SHARED((N_PAD, D), jnp.float32)]
            + [pltpu.VMEM((2, NB, WIN), jnp.int32)] * 2
            + [pltpu.VMEM((WIN, D), jnp.float32)] * NB
            + [pltpu.SemaphoreType.DMA] * (2 * NB + 3)
        ),
    )


def _sc_segment_sum(s, src, dst):
    npad = E_PAD - E
    pad_src = jnp.arange(npad, dtype=jnp.int32) % N
    pad_dst = jnp.arange(npad, dtype=jnp.int32) % (N_PAD - N - 8) + N
    src3 = jnp.concatenate([src, pad_src]).reshape(NW, NCHUNK, NB, WIN)
    dst3 = jnp.concatenate([dst, pad_dst]).reshape(NW, NCHUNK, NB, WIN)
    return _sc_segment_sum_kernel()(s, src3, dst3)


ROWB = 1000


def _tc_mm_body(x_ref, w_ref, o_ref):
    o_ref[...] = jnp.dot(x_ref[...], w_ref[...],
                         preferred_element_type=jnp.float32)


def _tc_mm(x, w):
    return pl.pallas_call(
        _tc_mm_body,
        grid=(N // ROWB,),
        in_specs=[
            pl.BlockSpec((ROWB, D), lambda i: (i, 0)),
            pl.BlockSpec((D, D), lambda i: (0, 0)),
        ],
        out_specs=pl.BlockSpec((ROWB, D), lambda i: (i, 0)),
        out_shape=jax.ShapeDtypeStruct((N, D), jnp.float32),
    )(x, w)


def _tc_combine_body(relu, p_ref, x_ref, w_ref, wl_ref, b_ref, hres_ref,
                     h_ref):
    t = (jnp.dot(p_ref[0] + p_ref[1], w_ref[...],
                 preferred_element_type=jnp.float32)
         + jnp.dot(x_ref[...], wl_ref[...], preferred_element_type=jnp.float32)
         + b_ref[0])
    if relu:
        t = jnp.maximum(t, 0.0)
    if hres_ref is not None:
        t = (hres_ref[...] + t) * 0.5
    h_ref[...] = t


def _tc_combine(p, x, w, wl, b, h_res, relu):
    has_res = h_res is not None
    if has_res:
        body = functools.partial(_tc_combine_body, relu)
    else:
        body = lambda p_, x_, w_, wl_, b_, h_: _tc_combine_body(
            relu, p_, x_, w_, wl_, b_, None, h_)
    in_specs = [
        pl.BlockSpec((NC, ROWB, D), lambda i: (0, i, 0)),
        pl.BlockSpec((ROWB, D), lambda i: (i, 0)),
        pl.BlockSpec((D, D), lambda i: (0, 0)),
        pl.BlockSpec((D, D), lambda i: (0, 0)),
        pl.BlockSpec((1, D), lambda i: (0, 0)),
    ]
    args = [p, x, w, wl, b.reshape(1, D)]
    if has_res:
        in_specs.append(pl.BlockSpec((ROWB, D), lambda i: (i, 0)))
        args.append(h_res)
    return pl.pallas_call(
        body,
        grid=(N // ROWB,),
        in_specs=in_specs,
        out_specs=pl.BlockSpec((ROWB, D), lambda i: (i, 0)),
        out_shape=jax.ShapeDtypeStruct((N, D), jnp.float32),
    )(*args)


def kernel(inputs, edge_index, W, Wl, b):
    src = edge_index[0]
    dst = edge_index[1]

    p = _sc_segment_sum(inputs, src, dst)
    h = _tc_combine(p, inputs, W[0], Wl[0], b[0], None, True)

    for i in range(NBLOCKS):
        j = 1 + 2 * i
        blk_in = h
        p = _sc_segment_sum(h, src, dst)
        t = _tc_combine(p, h, W[j], Wl[j], b[j], None, True)
        p = _sc_segment_sum(t, src, dst)
        h = _tc_combine(p, t, W[j + 1], Wl[j + 1], b[j + 1], blk_in, True)

    p = _sc_segment_sum(h, src, dst)
    x_out = _tc_combine(p, h, W[NCONVS - 1], Wl[NCONVS - 1], b[NCONVS - 1],
                        None, False)
    return (x_out, h)


# speedup vs baseline: 1.0482x; 1.0103x over previous
import functools

import jax
import jax.numpy as jnp
from jax import lax
from jax.experimental import pallas as pl
from jax.experimental.pallas import tpu as pltpu
from jax.experimental.pallas import tpu_sc as plsc

N = 10000
E = 320000
D = 128
NBLOCKS = 3
NCONVS = 2 + 2 * NBLOCKS

NC = 2
NS = 16
NW = NC * NS
WIN = 64
NB = 5
NWIN = 160
NCHUNK = NWIN // NB
EPW = NWIN * WIN
E_PAD = NW * EPW
N_PAD = 10240
RPW = N_PAD // NS
ZROWS = WIN


def _sc_segment_sum_body(support, src3, dst3, out, acc, srcbuf,
                         dstbuf, *bufs):
    rows = bufs[0:NB]
    gsem = bufs[NB:2 * NB]
    ssem = bufs[2 * NB:3 * NB]
    isem0, dsem, wsem = bufs[3 * NB:3 * NB + 3]

    c = lax.axis_index("c")
    s = lax.axis_index("s")
    wid = c * NS + s

    pltpu.async_copy(src3.at[wid, 0], srcbuf.at[0], isem0)
    pltpu.async_copy(dst3.at[wid, 0], dstbuf.at[0], dsem)

    zbuf = rows[0]

    def _zloop(i, carry):
        for j in range(D // 16):
            zbuf[i, pl.ds(j * 16, 16)] = jnp.zeros((16,), jnp.float32)
        return carry

    lax.fori_loop(0, ZROWS, _zloop, 0, unroll=False)
    zdescs = [pltpu.async_copy(
        zbuf, acc.at[pl.ds(s * RPW + r * ZROWS, ZROWS)], wsem)
        for r in range(RPW // ZROWS)]
    for d_ in zdescs:
        d_.wait()
    pltpu.make_async_copy(src3.at[wid, 0], srcbuf.at[0], isem0).wait()
    for b in range(NB):
        pltpu.async_copy(support.at[srcbuf.at[0, b]], rows[b], gsem[b])
    plsc.subcore_barrier()

    def _chunk(g, carry):
        w0 = g * NB
        p = lax.rem(g, 2)
        pn = lax.rem(g + 1, 2)
        pp = lax.rem(g + 1, 2)
        @pl.when(g > 0)
        def _wait_src_idx():
            pltpu.make_async_copy(
                src3.at[wid, g], srcbuf.at[p], isem0).wait()

        pltpu.make_async_copy(
            dst3.at[wid, g], dstbuf.at[p], dsem).wait()

        for b in range(NB):
            w = w0 + b

            @pl.when(g > 0)
            def _drain_and_gather():
                pltpu.make_async_copy(
                    rows[b], acc.at[dstbuf.at[pp, b]], ssem[b]).wait()
                pltpu.async_copy(
                    support.at[srcbuf.at[p, b]], rows[b], gsem[b])

        @pl.when(g + 1 < NCHUNK)
        def _prefetch():
            pltpu.async_copy(
                src3.at[wid, g + 1], srcbuf.at[pn], isem0)
            pltpu.async_copy(
                dst3.at[wid, g + 1], dstbuf.at[pn], dsem)

        for b in range(NB):
            w = w0 + b
            pltpu.make_async_copy(
                support.at[srcbuf.at[p, b]], rows[b], gsem[b]).wait()
            pltpu.async_copy(rows[b], acc.at[dstbuf.at[p, b]], ssem[b],
                             add=True)
        return carry

    lax.fori_loop(0, NCHUNK, _chunk, 0, unroll=False)
    pl_last = (NCHUNK - 1) % 2
    for b in range(NB):
        pltpu.make_async_copy(
            rows[b], acc.at[dstbuf.at[pl_last, b]], ssem[b]).wait()
    plsc.subcore_barrier()

    wdescs = []
    for r in range(RPW // ZROWS):
        row0 = s * RPW + r * ZROWS
        wdescs.append(pltpu.async_copy(
            acc.at[pl.ds(row0, ZROWS)], out.at[c, pl.ds(row0, ZROWS)], wsem))
    for d_ in wdescs:
        d_.wait()


@functools.cache
def _sc_segment_sum_kernel():
    return pl.kernel(
        _sc_segment_sum_body,
        out_type=jax.ShapeDtypeStruct((NC, N_PAD, D), jnp.float32),
        mesh=plsc.VectorSubcoreMesh(core_axis_name="c", subcore_axis_name="s",
                                    num_cores=NC, num_subcores=NS),
        scratch_types=(
            [pltpu.VMEM_SHARED((N_PAD, D), jnp.float32)]
            + [pltpu.VMEM((2, NB, WIN), jnp.int32)] * 2
            + [pltpu.VMEM((WIN, D), jnp.float32)] * NB
            + [pltpu.SemaphoreType.DMA] * (2 * NB + 3)
        ),
    )


def _sc_segment_sum(s, src, dst):
    npad = E_PAD - E
    pad_src = jnp.arange(npad, dtype=jnp.int32) % N
    pad_dst = jnp.arange(npad, dtype=jnp.int32) % (N_PAD - N - 8) + N
    src3 = jnp.concatenate([src, pad_src]).reshape(NW, NCHUNK, NB, WIN)
    dst3 = jnp.concatenate([dst, pad_dst]).reshape(NW, NCHUNK, NB, WIN)
    return _sc_segment_sum_kernel()(s, src3, dst3)


ROWB = 1000


def _tc_mm_body(x_ref, w_ref, o_ref):
    o_ref[...] = jnp.dot(x_ref[...], w_ref[...],
                         preferred_element_type=jnp.float32)


def _tc_mm(x, w):
    return pl.pallas_call(
        _tc_mm_body,
        grid=(N // ROWB,),
        in_specs=[
            pl.BlockSpec((ROWB, D), lambda i: (i, 0)),
            pl.BlockSpec((D, D), lambda i: (0, 0)),
        ],
        out_specs=pl.BlockSpec((ROWB, D), lambda i: (i, 0)),
        out_shape=jax.ShapeDtypeStruct((N, D), jnp.float32),
    )(x, w)


def _tc_combine_body(relu, p_ref, x_ref, w_ref, wl_ref, b_ref, hres_ref,
                     h_ref):
    t = (jnp.dot(p_ref[0] + p_ref[1], w_ref[...],
                 preferred_element_type=jnp.float32)
         + jnp.dot(x_ref[...], wl_ref[...], preferred_element_type=jnp.float32)
         + b_ref[0])
    if relu:
        t = jnp.maximum(t, 0.0)
    if hres_ref is not None:
        t = (hres_ref[...] + t) * 0.5
    h_ref[...] = t


def _tc_combine(p, x, w, wl, b, h_res, relu):
    has_res = h_res is not None
    if has_res:
        body = functools.partial(_tc_combine_body, relu)
    else:
        body = lambda p_, x_, w_, wl_, b_, h_: _tc_combine_body(
            relu, p_, x_, w_, wl_, b_, None, h_)
    in_specs = [
        pl.BlockSpec((NC, ROWB, D), lambda i: (0, i, 0)),
        pl.BlockSpec((ROWB, D), lambda i: (i, 0)),
        pl.BlockSpec((D, D), lambda i: (0, 0)),
        pl.BlockSpec((D, D), lambda i: (0, 0)),
        pl.BlockSpec((1, D), lambda i: (0, 0)),
    ]
    args = [p, x, w, wl, b.reshape(1, D)]
    if has_res:
        in_specs.append(pl.BlockSpec((ROWB, D), lambda i: (i, 0)))
        args.append(h_res)
    return pl.pallas_call(
        body,
        grid=(N // ROWB,),
        in_specs=in_specs,
        out_specs=pl.BlockSpec((ROWB, D), lambda i: (i, 0)),
        out_shape=jax.ShapeDtypeStruct((N, D), jnp.float32),
    )(*args)


def kernel(inputs, edge_index, W, Wl, b):
    src = edge_index[0]
    dst = edge_index[1]

    p = _sc_segment_sum(inputs, src, dst)
    h = _tc_combine(p, inputs, W[0], Wl[0], b[0], None, True)

    for i in range(NBLOCKS):
        j = 1 + 2 * i
        blk_in = h
        p = _sc_segment_sum(h, src, dst)
        t = _tc_combine(p, h, W[j], Wl[j], b[j], None, True)
        p = _sc_segment_sum(t, src, dst)
        h = _tc_combine(p, t, W[j + 1], Wl[j + 1], b[j + 1], blk_in, True)

    p = _sc_segment_sum(h, src, dst)
    x_out = _tc_combine(p, h, W[NCONVS - 1], Wl[NCONVS - 1], b[NCONVS - 1],
                        None, False)
    return (x_out, h)


# speedup vs baseline: 1.0715x; 1.0222x over previous
import functools

import jax
import jax.numpy as jnp
from jax import lax
from jax.experimental import pallas as pl
from jax.experimental.pallas import tpu as pltpu
from jax.experimental.pallas import tpu_sc as plsc

N = 10000
E = 320000
D = 128
NBLOCKS = 3
NCONVS = 2 + 2 * NBLOCKS

NC = 2
NS = 16
NW = NC * NS
WIN = 64
NB = 5
NWIN = 160
NCHUNK = NWIN // NB
EPW = NWIN * WIN
E_PAD = NW * EPW
N_PAD = 10240
RPW = N_PAD // NS
ZROWS = WIN


def _sc_segment_sum_body(support, src3, dst3, out, acc, srcbuf,
                         dstbuf, *bufs):
    rows = bufs[0:NB]
    gsem = bufs[NB:2 * NB]
    ssem = bufs[2 * NB:3 * NB]
    isem0, dsem, wsem = bufs[3 * NB:3 * NB + 3]

    c = lax.axis_index("c")
    s = lax.axis_index("s")
    wid = c * NS + s

    pltpu.async_copy(src3.at[wid, 0], srcbuf.at[0], isem0)
    pltpu.async_copy(dst3.at[wid, 0], dstbuf.at[0], dsem)

    zbuf = rows[0]

    def _zloop(i, carry):
        for j in range(D // 16):
            zbuf[i, pl.ds(j * 16, 16)] = jnp.zeros((16,), jnp.float32)
        return carry

    lax.fori_loop(0, ZROWS, _zloop, 0, unroll=False)
    zdescs = [pltpu.async_copy(
        zbuf, acc.at[pl.ds(s * RPW + r * ZROWS, ZROWS)], wsem)
        for r in range(RPW // ZROWS)]
    for d_ in zdescs:
        d_.wait()
    pltpu.make_async_copy(src3.at[wid, 0], srcbuf.at[0], isem0).wait()
    for b in range(NB):
        pltpu.async_copy(support.at[srcbuf.at[0, b]], rows[b], gsem[b])
    plsc.subcore_barrier()

    def _chunk(g, carry):
        w0 = g * NB
        p = lax.rem(g, 2)
        pn = lax.rem(g + 1, 2)
        pp = lax.rem(g + 1, 2)
        @pl.when(g > 0)
        def _wait_src_idx():
            pltpu.make_async_copy(
                src3.at[wid, g], srcbuf.at[p], isem0).wait()

        pltpu.make_async_copy(
            dst3.at[wid, g], dstbuf.at[p], dsem).wait()

        for b in range(NB):
            w = w0 + b

            @pl.when(g > 0)
            def _drain_and_gather():
                pltpu.make_async_copy(
                    rows[b], acc.at[dstbuf.at[pp, b]], ssem[b]).wait()
                pltpu.async_copy(
                    support.at[srcbuf.at[p, b]], rows[b], gsem[b])

        @pl.when(g + 1 < NCHUNK)
        def _prefetch():
            pltpu.async_copy(
                src3.at[wid, g + 1], srcbuf.at[pn], isem0)
            pltpu.async_copy(
                dst3.at[wid, g + 1], dstbuf.at[pn], dsem)

        for b in range(NB):
            w = w0 + b
            pltpu.make_async_copy(
                support.at[srcbuf.at[p, b]], rows[b], gsem[b]).wait()
            pltpu.async_copy(rows[b], acc.at[dstbuf.at[p, b]], ssem[b],
                             add=True)
        return carry

    lax.fori_loop(0, NCHUNK, _chunk, 0, unroll=False)
    pl_last = (NCHUNK - 1) % 2
    for b in range(NB):
        pltpu.make_async_copy(
            rows[b], acc.at[dstbuf.at[pl_last, b]], ssem[b]).wait()
    plsc.subcore_barrier()

    wdescs = []
    for r in range(RPW // ZROWS):
        row0 = s * RPW + r * ZROWS
        wdescs.append(pltpu.async_copy(
            acc.at[pl.ds(row0, ZROWS)], out.at[c, pl.ds(row0, ZROWS)], wsem))
    for d_ in wdescs:
        d_.wait()


@functools.cache
def _sc_segment_sum_kernel():
    return pl.kernel(
        _sc_segment_sum_body,
        out_type=jax.ShapeDtypeStruct((NC, N_PAD, D), jnp.float32),
        mesh=plsc.VectorSubcoreMesh(core_axis_name="c", subcore_axis_name="s",
                                    num_cores=NC, num_subcores=NS),
        scratch_types=(
            [pltpu.VMEM_SHARED((N_PAD, D), jnp.float32)]
            + [pltpu.VMEM((2, NB, WIN), jnp.int32)] * 2
            + [pltpu.VMEM((WIN, D), jnp.float32)] * NB
            + [pltpu.SemaphoreType.DMA] * (2 * NB + 3)
        ),
    )


def _sc_segment_sum(s, src, dst):
    npad = E_PAD - E
    pad_src = jnp.arange(npad, dtype=jnp.int32) % N
    pad_dst = jnp.arange(npad, dtype=jnp.int32) % (N_PAD - N - 8) + N
    src3 = jnp.concatenate([src, pad_src]).reshape(NW, NCHUNK, NB, WIN)
    dst3 = jnp.concatenate([dst, pad_dst]).reshape(NW, NCHUNK, NB, WIN)
    return _sc_segment_sum_kernel()(s, src3, dst3)


ROWB = 2000


def _tc_mm_body(x_ref, w_ref, o_ref):
    o_ref[...] = jnp.dot(x_ref[...], w_ref[...],
                         preferred_element_type=jnp.float32)


def _tc_mm(x, w):
    return pl.pallas_call(
        _tc_mm_body,
        grid=(N // ROWB,),
        in_specs=[
            pl.BlockSpec((ROWB, D), lambda i: (i, 0)),
            pl.BlockSpec((D, D), lambda i: (0, 0)),
        ],
        out_specs=pl.BlockSpec((ROWB, D), lambda i: (i, 0)),
        out_shape=jax.ShapeDtypeStruct((N, D), jnp.float32),
    )(x, w)


def _tc_combine_body(relu, p_ref, x_ref, w_ref, wl_ref, b_ref, hres_ref,
                     h_ref):
    t = (jnp.dot(p_ref[0] + p_ref[1], w_ref[...],
                 preferred_element_type=jnp.float32)
         + jnp.dot(x_ref[...], wl_ref[...], preferred_element_type=jnp.float32)
         + b_ref[0])
    if relu:
        t = jnp.maximum(t, 0.0)
    if hres_ref is not None:
        t = (hres_ref[...] + t) * 0.5
    h_ref[...] = t


def _tc_combine(p, x, w, wl, b, h_res, relu):
    has_res = h_res is not None
    if has_res:
        body = functools.partial(_tc_combine_body, relu)
    else:
        body = lambda p_, x_, w_, wl_, b_, h_: _tc_combine_body(
            relu, p_, x_, w_, wl_, b_, None, h_)
    in_specs = [
        pl.BlockSpec((NC, ROWB, D), lambda i: (0, i, 0)),
        pl.BlockSpec((ROWB, D), lambda i: (i, 0)),
        pl.BlockSpec((D, D), lambda i: (0, 0)),
        pl.BlockSpec((D, D), lambda i: (0, 0)),
        pl.BlockSpec((1, D), lambda i: (0, 0)),
    ]
    args = [p, x, w, wl, b.reshape(1, D)]
    if has_res:
        in_specs.append(pl.BlockSpec((ROWB, D), lambda i: (i, 0)))
        args.append(h_res)
    return pl.pallas_call(
        body,
        grid=(N // ROWB,),
        in_specs=in_specs,
        out_specs=pl.BlockSpec((ROWB, D), lambda i: (i, 0)),
        out_shape=jax.ShapeDtypeStruct((N, D), jnp.float32),
    )(*args)


def kernel(inputs, edge_index, W, Wl, b):
    src = edge_index[0]
    dst = edge_index[1]

    p = _sc_segment_sum(inputs, src, dst)
    h = _tc_combine(p, inputs, W[0], Wl[0], b[0], None, True)

    for i in range(NBLOCKS):
        j = 1 + 2 * i
        blk_in = h
        p = _sc_segment_sum(h, src, dst)
        t = _tc_combine(p, h, W[j], Wl[j], b[j], None, True)
        p = _sc_segment_sum(t, src, dst)
        h = _tc_combine(p, t, W[j + 1], Wl[j + 1], b[j + 1], blk_in, True)

    p = _sc_segment_sum(h, src, dst)
    x_out = _tc_combine(p, h, W[NCONVS - 1], Wl[NCONVS - 1], b[NCONVS - 1],
                        None, False)
    return (x_out, h)


# speedup vs baseline: 1.0785x; 1.0065x over previous
import functools

import jax
import jax.numpy as jnp
from jax import lax
from jax.experimental import pallas as pl
from jax.experimental.pallas import tpu as pltpu
from jax.experimental.pallas import tpu_sc as plsc

N = 10000
E = 320000
D = 128
NBLOCKS = 3
NCONVS = 2 + 2 * NBLOCKS

NC = 2
NS = 16
NW = NC * NS
WIN = 64
NB = 5
NWIN = 160
NCHUNK = NWIN // NB
EPW = NWIN * WIN
E_PAD = NW * EPW
N_PAD = 10240
RPW = N_PAD // NS
ZROWS = WIN


def _sc_segment_sum_body(support, src3, dst3, out, acc, srcbuf,
                         dstbuf, *bufs):
    rows = bufs[0:NB]
    gsem = bufs[NB:2 * NB]
    ssem = bufs[2 * NB:3 * NB]
    isem0, dsem, wsem = bufs[3 * NB:3 * NB + 3]

    c = lax.axis_index("c")
    s = lax.axis_index("s")
    wid = c * NS + s

    pltpu.async_copy(src3.at[wid, 0], srcbuf.at[0], isem0)
    pltpu.async_copy(dst3.at[wid, 0], dstbuf.at[0], dsem)

    zbuf = rows[0]

    def _zloop(i, carry):
        for j in range(D // 16):
            zbuf[i, pl.ds(j * 16, 16)] = jnp.zeros((16,), jnp.float32)
        return carry

    lax.fori_loop(0, ZROWS, _zloop, 0, unroll=False)
    zdescs = [pltpu.async_copy(
        zbuf, acc.at[pl.ds(s * RPW + r * ZROWS, ZROWS)], wsem)
        for r in range(RPW // ZROWS)]
    for d_ in zdescs:
        d_.wait()
    pltpu.make_async_copy(src3.at[wid, 0], srcbuf.at[0], isem0).wait()
    for b in range(NB):
        pltpu.async_copy(support.at[srcbuf.at[0, b]], rows[b], gsem[b])
    plsc.subcore_barrier()

    def _chunk(g, carry):
        w0 = g * NB
        p = lax.rem(g, 2)
        pn = lax.rem(g + 1, 2)
        pp = lax.rem(g + 1, 2)
        @pl.when(g > 0)
        def _wait_src_idx():
            pltpu.make_async_copy(
                src3.at[wid, g], srcbuf.at[p], isem0).wait()

        pltpu.make_async_copy(
            dst3.at[wid, g], dstbuf.at[p], dsem).wait()

        for b in range(NB):
            w = w0 + b

            @pl.when(g > 0)
            def _drain_and_gather():
                pltpu.make_async_copy(
                    rows[b], acc.at[dstbuf.at[pp, b]], ssem[b]).wait()
                pltpu.async_copy(
                    support.at[srcbuf.at[p, b]], rows[b], gsem[b])

        @pl.when(g + 1 < NCHUNK)
        def _prefetch():
            pltpu.async_copy(
                src3.at[wid, g + 1], srcbuf.at[pn], isem0)
            pltpu.async_copy(
                dst3.at[wid, g + 1], dstbuf.at[pn], dsem)

        for b in range(NB):
            w = w0 + b
            pltpu.make_async_copy(
                support.at[srcbuf.at[p, b]], rows[b], gsem[b]).wait()
            pltpu.async_copy(rows[b], acc.at[dstbuf.at[p, b]], ssem[b],
                             add=True)
        return carry

    lax.fori_loop(0, NCHUNK, _chunk, 0, unroll=False)
    pl_last = (NCHUNK - 1) % 2
    for b in range(NB):
        pltpu.make_async_copy(
            rows[b], acc.at[dstbuf.at[pl_last, b]], ssem[b]).wait()
    plsc.subcore_barrier()

    wdescs = []
    for r in range(RPW // ZROWS):
        row0 = s * RPW + r * ZROWS
        wdescs.append(pltpu.async_copy(
            acc.at[pl.ds(row0, ZROWS)], out.at[c, pl.ds(row0, ZROWS)], wsem))
    for d_ in wdescs:
        d_.wait()


@functools.cache
def _sc_segment_sum_kernel():
    return pl.kernel(
        _sc_segment_sum_body,
        out_type=jax.ShapeDtypeStruct((NC, N_PAD, D), jnp.float32),
        mesh=plsc.VectorSubcoreMesh(core_axis_name="c", subcore_axis_name="s",
                                    num_cores=NC, num_subcores=NS),
        scratch_types=(
            [pltpu.VMEM_SHARED((N_PAD, D), jnp.float32)]
            + [pltpu.VMEM((2, NB, WIN), jnp.int32)] * 2
            + [pltpu.VMEM((WIN, D), jnp.float32)] * NB
            + [pltpu.SemaphoreType.DMA] * (2 * NB + 3)
        ),
    )


def _sc_segment_sum(s, src, dst):
    npad = E_PAD - E
    pad_src = jnp.arange(npad, dtype=jnp.int32) % N
    pad_dst = jnp.arange(npad, dtype=jnp.int32) % (N_PAD - N - 8) + N
    src3 = jnp.concatenate([src, pad_src]).reshape(NW, NCHUNK, NB, WIN)
    dst3 = jnp.concatenate([dst, pad_dst]).reshape(NW, NCHUNK, NB, WIN)
    return _sc_segment_sum_kernel()(s, src3, dst3)


ROWB = 5000


def _tc_mm_body(x_ref, w_ref, o_ref):
    o_ref[...] = jnp.dot(x_ref[...], w_ref[...],
                         preferred_element_type=jnp.float32)


def _tc_mm(x, w):
    return pl.pallas_call(
        _tc_mm_body,
        grid=(N // ROWB,),
        in_specs=[
            pl.BlockSpec((ROWB, D), lambda i: (i, 0)),
            pl.BlockSpec((D, D), lambda i: (0, 0)),
        ],
        out_specs=pl.BlockSpec((ROWB, D), lambda i: (i, 0)),
        out_shape=jax.ShapeDtypeStruct((N, D), jnp.float32),
    )(x, w)


def _tc_combine_body(relu, p_ref, x_ref, w_ref, wl_ref, b_ref, hres_ref,
                     h_ref):
    t = (jnp.dot(p_ref[0] + p_ref[1], w_ref[...],
                 preferred_element_type=jnp.float32)
         + jnp.dot(x_ref[...], wl_ref[...], preferred_element_type=jnp.float32)
         + b_ref[0])
    if relu:
        t = jnp.maximum(t, 0.0)
    if hres_ref is not None:
        t = (hres_ref[...] + t) * 0.5
    h_ref[...] = t


def _tc_combine(p, x, w, wl, b, h_res, relu):
    has_res = h_res is not None
    if has_res:
        body = functools.partial(_tc_combine_body, relu)
    else:
        body = lambda p_, x_, w_, wl_, b_, h_: _tc_combine_body(
            relu, p_, x_, w_, wl_, b_, None, h_)
    in_specs = [
        pl.BlockSpec((NC, ROWB, D), lambda i: (0, i, 0)),
        pl.BlockSpec((ROWB, D), lambda i: (i, 0)),
        pl.BlockSpec((D, D), lambda i: (0, 0)),
        pl.BlockSpec((D, D), lambda i: (0, 0)),
        pl.BlockSpec((1, D), lambda i: (0, 0)),
    ]
    args = [p, x, w, wl, b.reshape(1, D)]
    if has_res:
        in_specs.append(pl.BlockSpec((ROWB, D), lambda i: (i, 0)))
        args.append(h_res)
    return pl.pallas_call(
        body,
        grid=(N // ROWB,),
        in_specs=in_specs,
        out_specs=pl.BlockSpec((ROWB, D), lambda i: (i, 0)),
        out_shape=jax.ShapeDtypeStruct((N, D), jnp.float32),
    )(*args)


def kernel(inputs, edge_index, W, Wl, b):
    src = edge_index[0]
    dst = edge_index[1]

    p = _sc_segment_sum(inputs, src, dst)
    h = _tc_combine(p, inputs, W[0], Wl[0], b[0], None, True)

    for i in range(NBLOCKS):
        j = 1 + 2 * i
        blk_in = h
        p = _sc_segment_sum(h, src, dst)
        t = _tc_combine(p, h, W[j], Wl[j], b[j], None, True)
        p = _sc_segment_sum(t, src, dst)
        h = _tc_combine(p, t, W[j + 1], Wl[j + 1], b[j + 1], blk_in, True)

    p = _sc_segment_sum(h, src, dst)
    x_out = _tc_combine(p, h, W[NCONVS - 1], Wl[NCONVS - 1], b[NCONVS - 1],
                        None, False)
    return (x_out, h)
